# baseline (speedup 1.0000x reference)
import functools

import jax
import jax.numpy as jnp
from jax import lax
from jax.experimental import pallas as pl
from jax.experimental.pallas import tpu as pltpu
from jax.experimental.pallas import tpu_sc as plsc

N = 10000
E = 320000
H = 128
DE = 4
L = 2
NC = 4
CS = 32
NOPS = 7
OUT = 10
GW = NC * NOPS * CS
NG = NC * NOPS

NCORE = 2
NSUB = 16
NW = NCORE * NSUB
ECH = 128
EPT = 10240
NCHUNK = EPT // ECH
E2 = NW * EPT
SLAB = 4
NSLAB = NCHUNK // SLAB
NP = 10240
RPT = NP // NSUB

C0 = 40
C1 = 2 * NCHUNK - C0
BLK = 400
NBLK = N // BLK

_f32 = jnp.float32


def _sc_h_body(h_hbm, idx_hbm, zh_hbm, outh,
               idx_v, rows_v, acc_h, sem_a, sem_b):
    cid = lax.axis_index("c")
    sid = lax.axis_index("s")
    row0 = sid * RPT
    start_chunk = jnp.where(cid == 0, sid * C0, NSUB * C0 + sid * C1)
    npairs = jnp.where(cid == 0, C0 // (2 * SLAB), C1 // (2 * SLAB))
    pltpu.sync_copy(zh_hbm.at[pl.ds(row0, RPT)], acc_h.at[pl.ds(row0, RPT)])
    plsc.subcore_barrier()

    def _rbuf(k):
        return rows_v.at[pl.ds(k * ECH, ECH)]

    sems = (sem_a, sem_b)
    CH8 = 2 * SLAB
    DEPTH = 2

    def _pair(p, carry):
        base = (start_chunk + p * CH8) * 2
        pltpu.sync_copy(idx_hbm.at[pl.ds(base, 2 * CH8)], idx_v)

        def _issue(j):
            return pltpu.async_copy(h_hbm.at[idx_v.at[2 * j]],
                                    _rbuf(j % DEPTH), sems[j % DEPTH])

        cps = [None] * CH8
        for j in range(DEPTH - 1):
            cps[j] = _issue(j)
        for j in range(CH8):
            if j + DEPTH - 1 < CH8:
                cps[j + DEPTH - 1] = _issue(j + DEPTH - 1)
            cps[j].wait()
            pltpu.sync_copy(_rbuf(j % DEPTH),
                            acc_h.at[idx_v.at[2 * j + 1]], add=True)
        return carry

    lax.fori_loop(0, npairs, _pair, 0)
    plsc.subcore_barrier()
    pltpu.sync_copy(acc_h.at[pl.ds(row0, RPT)], outh.at[cid, pl.ds(row0, RPT)])


def _sc_m_body(ea_hbm, idx_hbm, zh_hbm, outm,
               idx_v, rows_v, acc_m, sem_a, sem_b):
    cid = lax.axis_index("c")
    sid = lax.axis_index("s")
    wid = cid * NSUB + sid
    row0 = sid * RPT
    pltpu.sync_copy(zh_hbm.at[pl.ds(row0, RPT)], acc_m.at[pl.ds(row0, RPT)])
    plsc.subcore_barrier()

    def _rbuf(k):
        return rows_v.at[pl.ds(k * ECH, ECH)]

    sems = (sem_a, sem_b)
    CH8 = 2 * SLAB
    DEPTH = 2

    def _pair(p, carry):
        base = (wid * NCHUNK + p * CH8) * 2
        pltpu.sync_copy(idx_hbm.at[pl.ds(base, 2 * CH8)], idx_v)

        def _issue(j):
            erow = (wid * NCHUNK + p * CH8 + j) * ECH
            return pltpu.async_copy(ea_hbm.at[pl.ds(erow, ECH)],
                                    _rbuf(j % DEPTH), sems[j % DEPTH])

        cps = [None] * CH8
        for j in range(DEPTH - 1):
            cps[j] = _issue(j)
        for j in range(CH8):
            if j + DEPTH - 1 < CH8:
                cps[j + DEPTH - 1] = _issue(j + DEPTH - 1)
            cps[j].wait()
            pltpu.sync_copy(_rbuf(j % DEPTH),
                            acc_m.at[idx_v.at[2 * j + 1]], add=True)
        return carry

    lax.fori_loop(0, NSLAB // 2, _pair, 0)
    plsc.subcore_barrier()
    pltpu.sync_copy(acc_m.at[pl.ds(row0, RPT)], outm.at[cid, pl.ds(row0, RPT)])


_SC_MESH = plsc.VectorSubcoreMesh(core_axis_name="c", subcore_axis_name="s")

_SC_SCRATCH = [
    pltpu.VMEM((4 * SLAB, ECH), jnp.int32),
    pltpu.VMEM((2 * ECH, H), _f32),
    pltpu.VMEM_SHARED((NP, H), _f32),
    pltpu.SemaphoreType.DMA,
    pltpu.SemaphoreType.DMA,
]

_sc_scatter_h = functools.partial(
    pl.kernel,
    out_type=[jax.ShapeDtypeStruct((NCORE, NP, H), _f32)],
    mesh=_SC_MESH,
    scratch_types=_SC_SCRATCH,
)(_sc_h_body)

_sc_scatter_m = functools.partial(
    pl.kernel,
    out_type=[jax.ShapeDtypeStruct((NCORE, NP, H), _f32)],
    mesh=_SC_MESH,
    scratch_types=_SC_SCRATCH,
)(_sc_m_body)


def _proj_body(x_ref, w_ref, b_ref, o_ref):
    o_ref[...] = jnp.dot(x_ref[...], w_ref[...],
                         preferred_element_type=_f32) + b_ref[...]


def _mix_body(p_ref, m_ref, wcat_ref, wec_ref, bcat_ref, g_ref, bln_ref,
              wmix_ref, gmat_ref, gt_ref, sel_ref, e4_ref, hn_ref, st_ref):
    i = pl.program_id(0)
    P = p_ref[...]
    Sh = P[0] + P[1]
    Mp = m_ref[...]
    M = Mp[0] + Mp[1]
    deg = jnp.maximum(jnp.dot(M, e4_ref[...], preferred_element_type=_f32), 1.0)
    dinv = 1.0 / deg
    A = Sh * dinv
    Bm = M * dinv
    Rm = (jnp.dot(A, wcat_ref[...], preferred_element_type=_f32)
          + jnp.dot(Bm, wec_ref[...], preferred_element_type=_f32)
          + bcat_ref[...])
    mu = jnp.dot(Rm, gmat_ref[...], preferred_element_type=_f32) * (1.0 / CS)
    mq = jnp.dot(Rm * Rm, gmat_ref[...], preferred_element_type=_f32) * (1.0 / CS)
    mu_b = jnp.dot(mu, gt_ref[...], preferred_element_type=_f32)
    var_b = jnp.dot(mq, gt_ref[...], preferred_element_type=_f32) - mu_b * mu_b
    resn = (Rm - mu_b) * lax.rsqrt(var_b + 1e-5) * g_ref[...] + bln_ref[...]
    hn = jnp.dot(resn * wmix_ref[...], sel_ref[...], preferred_element_type=_f32)
    hn_ref[...] = hn

    @pl.when(i == 0)
    def _():
        st_ref[...] = jnp.zeros_like(st_ref)

    st_ref[...] += jnp.concatenate(
        [jnp.sum(hn, 0, keepdims=True), jnp.sum(hn * hn, 0, keepdims=True)], axis=0)


def _bn_body(hn_ref, st_ref, h_ref):
    st = st_ref[...]
    mu = st[0:1] * (1.0 / N)
    var = st[1:2] * (1.0 / N) - mu * mu
    h_ref[...] = jnp.maximum((hn_ref[...] - mu) * lax.rsqrt(var + 1e-5), 0.0)


def _final_body(h0_ref, h1_ref, h2_ref, wt_ref, mt_ref, bc_ref, out_ref,
                ssum, smax):
    i = pl.program_id(0)

    @pl.when(i == 0)
    def _():
        ssum[...] = jnp.zeros_like(ssum)
        smax[...] = jnp.full_like(smax, -jnp.inf)

    hs = (h0_ref[...], h1_ref[...], h2_ref[...])
    for k in range(3):
        ssum[k:k + 1, :] += jnp.sum(hs[k], 0, keepdims=True)
        smax[k:k + 1, :] = jnp.maximum(smax[k:k + 1, :],
                                       jnp.max(hs[k], 0, keepdims=True))

    @pl.when(i == NBLK - 1)
    def _():
        Wm = wt_ref[...] * mt_ref[...]
        mean = ssum[...] * (1.0 / N)
        mx = smax[...]
        acc = bc_ref[...]
        for k in range(3):
            acc += jnp.dot(mean[k:k + 1, :], Wm[k], preferred_element_type=_f32)
        for k in range(3):
            acc += jnp.dot(mx[k:k + 1, :], Wm[k + 3], preferred_element_type=_f32)
        out_ref[...] = acc


def _const_spec(shape):
    return pl.BlockSpec(shape, lambda i: tuple(0 for _ in shape))


def _tc_proj(x, W_in, b_in2):
    return pl.pallas_call(
        _proj_body,
        grid=(NBLK,),
        in_specs=[pl.BlockSpec((BLK, H), lambda i: (i, 0)),
                  _const_spec((H, H)),
                  _const_spec((1, H))],
        out_specs=pl.BlockSpec((BLK, H), lambda i: (i, 0)),
        out_shape=jax.ShapeDtypeStruct((N, H), _f32),
    )(x, W_in, b_in2)


def _tc_mix(parts_h, parts_m, wcat, wec, bcat, gfull, bfull, wfull,
            gmat, gmatT, sel, e4):
    return pl.pallas_call(
        _mix_body,
        grid=(NBLK,),
        in_specs=[pl.BlockSpec((NCORE, BLK, H), lambda i: (0, i, 0)),
                  pl.BlockSpec((NCORE, BLK, H), lambda i: (0, i, 0)),
                  _const_spec((H, GW)),
                  _const_spec((H, GW)),
                  _const_spec((1, GW)),
                  _const_spec((1, GW)),
                  _const_spec((1, GW)),
                  _const_spec((1, GW)),
                  _const_spec((GW, NG)),
                  _const_spec((NG, GW)),
                  _const_spec((GW, NC * CS)),
                  _const_spec((H, 1))],
        out_specs=[pl.BlockSpec((BLK, H), lambda i: (i, 0)),
                   _const_spec((2, H))],
        out_shape=[jax.ShapeDtypeStruct((N, H), _f32),
                   jax.ShapeDtypeStruct((2, H), _f32)],
    )(parts_h, parts_m, wcat, wec, bcat, gfull, bfull, wfull, gmat, gmatT,
      sel, e4)


def _tc_bn(hn, st):
    return pl.pallas_call(
        _bn_body,
        grid=(NBLK,),
        in_specs=[pl.BlockSpec((BLK, H), lambda i: (i, 0)),
                  _const_spec((2, H))],
        out_specs=pl.BlockSpec((BLK, H), lambda i: (i, 0)),
        out_shape=jax.ShapeDtypeStruct((N, H), _f32),
    )(hn, st)


def _tc_final(h0, h1, h2, wT, mT, bc2):
    return pl.pallas_call(
        _final_body,
        grid=(NBLK,),
        in_specs=[pl.BlockSpec((BLK, H), lambda i: (i, 0)),
                  pl.BlockSpec((BLK, H), lambda i: (i, 0)),
                  pl.BlockSpec((BLK, H), lambda i: (i, 0)),
                  _const_spec((6, H, OUT)),
                  _const_spec((6, H, OUT)),
                  _const_spec((1, OUT))],
        out_specs=_const_spec((1, OUT)),
        out_shape=jax.ShapeDtypeStruct((1, OUT), _f32),
        scratch_shapes=[pltpu.VMEM((3, H), _f32), pltpu.VMEM((3, H), _f32)],
    )(h0, h1, h2, wT, mT, bc2)


def kernel(x, edge_index, edge_attr, W_in, b_in, We, Wops, bops, ln_g, ln_b,
           alpha, Wcls, mask, bcls):
    pad = E2 - E
    src_p = jnp.concatenate([edge_index[0], jnp.zeros((pad,), jnp.int32)])
    dst_p = jnp.concatenate(
        [edge_index[1], N + (jnp.arange(pad, dtype=jnp.int32) % (NP - N))])
    idx2 = jnp.stack([src_p.reshape(NW * NCHUNK, ECH),
                      dst_p.reshape(NW * NCHUNK, ECH)],
                     axis=1).reshape(NW * NCHUNK * 2, ECH)
    ea128 = jnp.concatenate(
        [jnp.concatenate([edge_attr, jnp.ones((E, 1), _f32),
                          jnp.zeros((E, H - DE - 1), _f32)], axis=1),
         jnp.zeros((pad, H), _f32)], axis=0)
    zh = jnp.zeros((NP, H), _f32)

    Wcat = Wops.transpose(0, 3, 1, 2, 4).reshape(L, H, GW)
    Wec = jnp.einsum('lodh,lcohs->lcods', We, Wops)
    Wec128 = jnp.concatenate(
        [Wec.transpose(0, 3, 1, 2, 4).reshape(L, DE, GW),
         jnp.zeros((L, H - DE, GW), _f32)], axis=1)
    bcat = bops.reshape(L, 1, GW)
    gfull = jnp.broadcast_to(ln_g[:, :, None, :], (L, NC, NOPS, CS)).reshape(L, 1, GW)
    bfull = jnp.broadcast_to(ln_b[:, :, None, :], (L, NC, NOPS, CS)).reshape(L, 1, GW)
    wfull = jnp.broadcast_to(jax.nn.softmax(alpha, -1)[..., None],
                             (L, NC, NOPS, CS)).reshape(L, 1, GW)
    ar = jnp.arange(GW)
    gmat = (ar[:, None] // CS == jnp.arange(NG)[None, :]).astype(_f32)
    gmatT = gmat.T
    cols = (ar // (NOPS * CS)) * CS + (ar % CS)
    sel = (cols[:, None] == jnp.arange(NC * CS)[None, :]).astype(_f32)
    e4 = (jnp.arange(H) == DE).astype(_f32)[:, None]
    wT = Wcls.T.reshape(2 * (L + 1), H, OUT)
    mT = mask.T.reshape(2 * (L + 1), H, OUT)
    bc2 = bcls[None, :]
    b_in2 = b_in[None, :]

    h0 = _tc_proj(x, W_in, b_in2)

    (parts_m,) = _sc_scatter_m(ea128, idx2, zh)
    (parts_h,) = _sc_scatter_h(h0, idx2, zh)
    hn0, st0 = _tc_mix(parts_h, parts_m, Wcat[0], Wec128[0], bcat[0], gfull[0],
                       bfull[0], wfull[0], gmat, gmatT, sel, e4)
    h1 = _tc_bn(hn0, st0)

    (parts_h2,) = _sc_scatter_h(h1, idx2, zh)
    hn1, st1 = _tc_mix(parts_h2, parts_m, Wcat[1], Wec128[1], bcat[1], gfull[1],
                       bfull[1], wfull[1], gmat, gmatT, sel, e4)
    h2 = _tc_bn(hn1, st1)

    out2 = _tc_final(h0, h1, h2, wT, mT, bc2)
    return out2[0]


# speedup vs baseline: 1.0504x; 1.0504x over previous
import functools

import jax
import jax.numpy as jnp
from jax import lax
from jax.experimental import pallas as pl
from jax.experimental.pallas import tpu as pltpu
from jax.experimental.pallas import tpu_sc as plsc

N = 10000
E = 320000
H = 128
DE = 4
L = 2
NC = 4
CS = 32
NOPS = 7
OUT = 10
GW = NC * NOPS * CS
NG = NC * NOPS

NCORE = 2
NSUB = 16
NW = NCORE * NSUB
ECH = 128
EPT = 10240
NCHUNK = EPT // ECH
E2 = NW * EPT
SLAB = 4
NSLAB = NCHUNK // SLAB
NP = 10240
RPT = NP // NSUB

BLK = 400
NBLK = N // BLK

_f32 = jnp.float32


def _sc_h_body(h_hbm, idx_hbm, zh_hbm, outh,
               idx_v, rows_v, acc_h, sem_a, sem_b):
    cid = lax.axis_index("c")
    sid = lax.axis_index("s")
    wid = cid * NSUB + sid
    row0 = sid * RPT
    pltpu.sync_copy(zh_hbm.at[pl.ds(row0, RPT)], acc_h.at[pl.ds(row0, RPT)])
    plsc.subcore_barrier()

    def _rbuf(k):
        return rows_v.at[pl.ds(k * ECH, ECH)]

    sems = (sem_a, sem_b)
    CH8 = 2 * SLAB
    DEPTH = 2

    def _pair(p, carry):
        base = (wid * NCHUNK + p * CH8) * 2
        pltpu.sync_copy(idx_hbm.at[pl.ds(base, 2 * CH8)], idx_v)

        def _issue(j):
            return pltpu.async_copy(h_hbm.at[idx_v.at[2 * j]],
                                    _rbuf(j % DEPTH), sems[j % DEPTH])

        cps = [None] * CH8
        for j in range(DEPTH - 1):
            cps[j] = _issue(j)
        for j in range(CH8):
            if j + DEPTH - 1 < CH8:
                cps[j + DEPTH - 1] = _issue(j + DEPTH - 1)
            cps[j].wait()
            pltpu.sync_copy(_rbuf(j % DEPTH),
                            acc_h.at[idx_v.at[2 * j + 1]], add=True)
        return carry

    lax.fori_loop(0, NSLAB // 2, _pair, 0)
    plsc.subcore_barrier()
    pltpu.sync_copy(acc_h.at[pl.ds(row0, RPT)], outh.at[cid, pl.ds(row0, RPT)])


def _sc_m_body(ea_hbm, idx_hbm, zh_hbm, outm,
               idx_v, rows_v, acc_m, sem_a, sem_b):
    cid = lax.axis_index("c")
    sid = lax.axis_index("s")
    wid = cid * NSUB + sid
    row0 = sid * RPT
    pltpu.sync_copy(zh_hbm.at[pl.ds(row0, RPT)], acc_m.at[pl.ds(row0, RPT)])
    plsc.subcore_barrier()

    def _rbuf(k):
        return rows_v.at[pl.ds(k * ECH, ECH)]

    sems = (sem_a, sem_b)
    CH8 = 2 * SLAB
    DEPTH = 2

    def _pair(p, carry):
        base = (wid * NCHUNK + p * CH8) * 2
        pltpu.sync_copy(idx_hbm.at[pl.ds(base, 2 * CH8)], idx_v)

        def _issue(j):
            erow = (wid * NCHUNK + p * CH8 + j) * ECH
            return pltpu.async_copy(ea_hbm.at[pl.ds(erow, ECH)],
                                    _rbuf(j % DEPTH), sems[j % DEPTH])

        cps = [None] * CH8
        for j in range(DEPTH - 1):
            cps[j] = _issue(j)
        for j in range(CH8):
            if j + DEPTH - 1 < CH8:
                cps[j + DEPTH - 1] = _issue(j + DEPTH - 1)
            cps[j].wait()
            pltpu.sync_copy(_rbuf(j % DEPTH),
                            acc_m.at[idx_v.at[2 * j + 1]], add=True)
        return carry

    lax.fori_loop(0, NSLAB // 2, _pair, 0)
    plsc.subcore_barrier()
    pltpu.sync_copy(acc_m.at[pl.ds(row0, RPT)], outm.at[cid, pl.ds(row0, RPT)])


_SC_MESH = plsc.VectorSubcoreMesh(core_axis_name="c", subcore_axis_name="s")

_SC_SCRATCH = [
    pltpu.VMEM((4 * SLAB, ECH), jnp.int32),
    pltpu.VMEM((2 * ECH, H), _f32),
    pltpu.VMEM_SHARED((NP, H), _f32),
    pltpu.SemaphoreType.DMA,
    pltpu.SemaphoreType.DMA,
]

_sc_scatter_h = functools.partial(
    pl.kernel,
    out_type=[jax.ShapeDtypeStruct((NCORE, NP, H), _f32)],
    mesh=_SC_MESH,
    scratch_types=_SC_SCRATCH,
)(_sc_h_body)

_sc_scatter_m = functools.partial(
    pl.kernel,
    out_type=[jax.ShapeDtypeStruct((NCORE, NP, H), _f32)],
    mesh=_SC_MESH,
    scratch_types=_SC_SCRATCH,
)(_sc_m_body)


def _proj_body(x_ref, w_ref, b_ref, o_ref):
    o_ref[...] = jnp.dot(x_ref[...], w_ref[...],
                         preferred_element_type=_f32) + b_ref[...]


def _mix_body(p_ref, m_ref, wcat_ref, wec_ref, bcat_ref, g_ref, bln_ref,
              wmix_ref, gmat_ref, gt_ref, sel_ref, e4_ref, hn_ref, st_ref):
    i = pl.program_id(0)
    P = p_ref[...]
    Sh = P[0] + P[1]
    Mp = m_ref[...]
    M = Mp[0] + Mp[1]
    deg = jnp.maximum(jnp.dot(M, e4_ref[...], preferred_element_type=_f32), 1.0)
    dinv = 1.0 / deg
    A = Sh * dinv
    Bm = M * dinv
    Rm = (jnp.dot(A, wcat_ref[...], preferred_element_type=_f32)
          + jnp.dot(Bm, wec_ref[...], preferred_element_type=_f32)
          + bcat_ref[...])
    mu = jnp.dot(Rm, gmat_ref[...], preferred_element_type=_f32) * (1.0 / CS)
    mq = jnp.dot(Rm * Rm, gmat_ref[...], preferred_element_type=_f32) * (1.0 / CS)
    mu_b = jnp.dot(mu, gt_ref[...], preferred_element_type=_f32)
    var_b = jnp.dot(mq, gt_ref[...], preferred_element_type=_f32) - mu_b * mu_b
    resn = (Rm - mu_b) * lax.rsqrt(var_b + 1e-5) * g_ref[...] + bln_ref[...]
    hn = jnp.dot(resn * wmix_ref[...], sel_ref[...], preferred_element_type=_f32)
    hn_ref[...] = hn

    @pl.when(i == 0)
    def _():
        st_ref[...] = jnp.zeros_like(st_ref)

    st_ref[...] += jnp.concatenate(
        [jnp.sum(hn, 0, keepdims=True), jnp.sum(hn * hn, 0, keepdims=True)], axis=0)


def _bn_body(hn_ref, st_ref, h_ref):
    st = st_ref[...]
    mu = st[0:1] * (1.0 / N)
    var = st[1:2] * (1.0 / N) - mu * mu
    h_ref[...] = jnp.maximum((hn_ref[...] - mu) * lax.rsqrt(var + 1e-5), 0.0)


def _final_body(h0_ref, h1_ref, hn_ref, st_ref, wt_ref, mt_ref, bc_ref,
                out_ref, ssum, smax):
    i = pl.program_id(0)

    @pl.when(i == 0)
    def _():
        ssum[...] = jnp.zeros_like(ssum)
        smax[...] = jnp.full_like(smax, -jnp.inf)

    st = st_ref[...]
    mu2 = st[0:1] * (1.0 / N)
    var2 = st[1:2] * (1.0 / N) - mu2 * mu2
    h2 = jnp.maximum((hn_ref[...] - mu2) * lax.rsqrt(var2 + 1e-5), 0.0)
    hs = (h0_ref[...], h1_ref[...], h2)
    for k in range(3):
        ssum[k:k + 1, :] += jnp.sum(hs[k], 0, keepdims=True)
        smax[k:k + 1, :] = jnp.maximum(smax[k:k + 1, :],
                                       jnp.max(hs[k], 0, keepdims=True))

    @pl.when(i == NBLK - 1)
    def _():
        Wm = wt_ref[...] * mt_ref[...]
        mean = ssum[...] * (1.0 / N)
        mx = smax[...]
        acc = bc_ref[...]
        for k in range(3):
            acc += jnp.dot(mean[k:k + 1, :], Wm[k], preferred_element_type=_f32)
        for k in range(3):
            acc += jnp.dot(mx[k:k + 1, :], Wm[k + 3], preferred_element_type=_f32)
        out_ref[...] = acc


def _const_spec(shape):
    return pl.BlockSpec(shape, lambda i: tuple(0 for _ in shape))


def _tc_proj(x, W_in, b_in2):
    return pl.pallas_call(
        _proj_body,
        grid=(NBLK,),
        in_specs=[pl.BlockSpec((BLK, H), lambda i: (i, 0)),
                  _const_spec((H, H)),
                  _const_spec((1, H))],
        out_specs=pl.BlockSpec((BLK, H), lambda i: (i, 0)),
        out_shape=jax.ShapeDtypeStruct((N, H), _f32),
    )(x, W_in, b_in2)


def _tc_mix(parts_h, parts_m, wcat, wec, bcat, gfull, bfull, wfull,
            gmat, gmatT, sel, e4):
    return pl.pallas_call(
        _mix_body,
        grid=(NBLK,),
        in_specs=[pl.BlockSpec((NCORE, BLK, H), lambda i: (0, i, 0)),
                  pl.BlockSpec((NCORE, BLK, H), lambda i: (0, i, 0)),
                  _const_spec((H, GW)),
                  _const_spec((H, GW)),
                  _const_spec((1, GW)),
                  _const_spec((1, GW)),
                  _const_spec((1, GW)),
                  _const_spec((1, GW)),
                  _const_spec((GW, NG)),
                  _const_spec((NG, GW)),
                  _const_spec((GW, NC * CS)),
                  _const_spec((H, 1))],
        out_specs=[pl.BlockSpec((BLK, H), lambda i: (i, 0)),
                   _const_spec((2, H))],
        out_shape=[jax.ShapeDtypeStruct((N, H), _f32),
                   jax.ShapeDtypeStruct((2, H), _f32)],
    )(parts_h, parts_m, wcat, wec, bcat, gfull, bfull, wfull, gmat, gmatT,
      sel, e4)


def _tc_bn(hn, st):
    return pl.pallas_call(
        _bn_body,
        grid=(NBLK,),
        in_specs=[pl.BlockSpec((BLK, H), lambda i: (i, 0)),
                  _const_spec((2, H))],
        out_specs=pl.BlockSpec((BLK, H), lambda i: (i, 0)),
        out_shape=jax.ShapeDtypeStruct((N, H), _f32),
    )(hn, st)


def _tc_final(h0, h1, hn1, st1, wT, mT, bc2):
    return pl.pallas_call(
        _final_body,
        grid=(NBLK,),
        in_specs=[pl.BlockSpec((BLK, H), lambda i: (i, 0)),
                  pl.BlockSpec((BLK, H), lambda i: (i, 0)),
                  pl.BlockSpec((BLK, H), lambda i: (i, 0)),
                  _const_spec((2, H)),
                  _const_spec((6, H, OUT)),
                  _const_spec((6, H, OUT)),
                  _const_spec((1, OUT))],
        out_specs=_const_spec((1, OUT)),
        out_shape=jax.ShapeDtypeStruct((1, OUT), _f32),
        scratch_shapes=[pltpu.VMEM((3, H), _f32), pltpu.VMEM((3, H), _f32)],
    )(h0, h1, hn1, st1, wT, mT, bc2)


def kernel(x, edge_index, edge_attr, W_in, b_in, We, Wops, bops, ln_g, ln_b,
           alpha, Wcls, mask, bcls):
    pad = E2 - E
    src_p = jnp.concatenate([edge_index[0], jnp.zeros((pad,), jnp.int32)])
    dst_p = jnp.concatenate(
        [edge_index[1], N + (jnp.arange(pad, dtype=jnp.int32) % (NP - N))])
    idx2 = jnp.stack([src_p.reshape(NW * NCHUNK, ECH),
                      dst_p.reshape(NW * NCHUNK, ECH)],
                     axis=1).reshape(NW * NCHUNK * 2, ECH)
    ea128 = jnp.concatenate(
        [jnp.concatenate([edge_attr, jnp.ones((E, 1), _f32),
                          jnp.zeros((E, H - DE - 1), _f32)], axis=1),
         jnp.zeros((pad, H), _f32)], axis=0)
    zh = jnp.zeros((NP, H), _f32)

    Wcat = Wops.transpose(0, 3, 1, 2, 4).reshape(L, H, GW)
    Wec = jnp.einsum('lodh,lcohs->lcods', We, Wops)
    Wec128 = jnp.concatenate(
        [Wec.transpose(0, 3, 1, 2, 4).reshape(L, DE, GW),
         jnp.zeros((L, H - DE, GW), _f32)], axis=1)
    bcat = bops.reshape(L, 1, GW)
    gfull = jnp.broadcast_to(ln_g[:, :, None, :], (L, NC, NOPS, CS)).reshape(L, 1, GW)
    bfull = jnp.broadcast_to(ln_b[:, :, None, :], (L, NC, NOPS, CS)).reshape(L, 1, GW)
    wfull = jnp.broadcast_to(jax.nn.softmax(alpha, -1)[..., None],
                             (L, NC, NOPS, CS)).reshape(L, 1, GW)
    ar = jnp.arange(GW)
    gmat = (ar[:, None] // CS == jnp.arange(NG)[None, :]).astype(_f32)
    gmatT = gmat.T
    cols = (ar // (NOPS * CS)) * CS + (ar % CS)
    sel = (cols[:, None] == jnp.arange(NC * CS)[None, :]).astype(_f32)
    e4 = (jnp.arange(H) == DE).astype(_f32)[:, None]
    wT = Wcls.T.reshape(2 * (L + 1), H, OUT)
    mT = mask.T.reshape(2 * (L + 1), H, OUT)
    bc2 = bcls[None, :]
    b_in2 = b_in[None, :]

    h0 = _tc_proj(x, W_in, b_in2)

    (parts_m,) = _sc_scatter_m(ea128, idx2, zh)
    (parts_h,) = _sc_scatter_h(h0, idx2, zh)
    hn0, st0 = _tc_mix(parts_h, parts_m, Wcat[0], Wec128[0], bcat[0], gfull[0],
                       bfull[0], wfull[0], gmat, gmatT, sel, e4)
    h1 = _tc_bn(hn0, st0)

    (parts_h2,) = _sc_scatter_h(h1, idx2, zh)
    hn1, st1 = _tc_mix(parts_h2, parts_m, Wcat[1], Wec128[1], bcat[1], gfull[1],
                       bfull[1], wfull[1], gmat, gmatT, sel, e4)

    out2 = _tc_final(h0, h1, hn1, st1, wT, mT, bc2)
    return out2[0]
